# baseline (speedup 1.0000x reference)
import functools

import jax
import jax.numpy as jnp
from jax import lax
from jax.experimental import pallas as pl
from jax.experimental.pallas import tpu as pltpu
from jax.experimental.pallas import tpu_sc as plsc

_SMOOTH = 1e-05
_SENT = 0x7FFFFFFF
_N = 262144
_Q = _N // 4
_NV = _Q // 16


def _prep_kernel(logits_ref, target_ref, eps_ref, bits_ref, p_ref, nfg_ref):
    l = logits_ref[0]
    d = l[1] - l[0]
    p = 1.0 / (1.0 + jnp.exp(-d))
    fg = target_ref[0] == 1
    tp = p * (jnp.where(fg, 1.0, 0.0) + eps_ref[0])
    bits = lax.bitcast_convert_type(tp, jnp.int32)
    bits_ref[0] = jnp.where(fg, bits, jnp.int32(_SENT))
    p_ref[0] = p
    nfg_ref[0] = jnp.full((8, 128), jnp.sum(fg.astype(jnp.int32)),
                          dtype=jnp.int32)


_sc_mesh = plsc.VectorSubcoreMesh(core_axis_name="c", subcore_axis_name="s")


@functools.partial(
    pl.kernel,
    mesh=_sc_mesh,
    compiler_params=pltpu.CompilerParams(needs_layout_passes=False),
    out_type=[
        jax.ShapeDtypeStruct((16 * 16,), jnp.float32),
        jax.ShapeDtypeStruct((64, 16), jnp.int32),
    ],
    scratch_types=[
        pltpu.VMEM((_Q + 16,), jnp.int32),
        pltpu.VMEM((16,), jnp.int32),
        pltpu.VMEM((4, 16), jnp.int32),
        pltpu.VMEM((16,), jnp.float32),
    ],
)
def _select_kernel(bits_hbm, p_hbm, nfg_hbm, dice_hbm, cx_hbm,
                   keys, mine, part, drow):
    c = lax.axis_index("c")
    s = lax.axis_index("s")
    quarter = s % 4
    g4 = (s // 4) * 4
    zero = jnp.zeros((16,), jnp.int32)

    def exchange(row_vec, slot):
        mine[...] = row_vec
        pltpu.sync_copy(mine, cx_hbm.at[slot * 32 + c * 16 + s])
        plsc.subcore_barrier()
        pltpu.sync_copy(cx_hbm.at[pl.ds(slot * 32 + c * 16 + g4, 4)], part)
        return part[0], part[1], part[2], part[3]

    def merged(cnt_vec, slot):
        r0, r1, r2, r3 = exchange(cnt_vec, slot)
        return r0 + r1 + r2 + r3

    def count_lt(cand_vec):
        @plsc.parallel_loop(0, _NV, unroll=8, carry=zero)
        def acc(i, a):
            x = keys[pl.ds(i * 16, 16)]
            return a + jnp.where(x < cand_vec, 1, 0)
        return jnp.broadcast_to(jnp.sum(acc), (16,))

    def count_lt_dyn(cand_vec, ntrip):
        def body(i, acc):
            x = keys[pl.ds(i * 16, 16)]
            return acc + jnp.where(x < cand_vec, 1, 0)
        acc = lax.fori_loop(0, ntrip, body, zero)
        return jnp.broadcast_to(jnp.sum(acc), (16,))

    for ph in range(2):
        sample = ph * 8 + c * 4 + s // 4
        base = sample * _N + quarter * _Q
        pltpu.sync_copy(bits_hbm.at[pl.ds(base, _Q)], keys.at[pl.ds(0, _Q)])
        pltpu.sync_copy(nfg_hbm.at[pl.ds(sample * 1024, 16)], mine)
        n_fg = mine[...]
        k = jnp.maximum(1, n_fg >> 1)

        def l1_iter(i, carry):
            res, below = carry
            cand = res | (jnp.int32(1) << (jnp.int32(29) - i))
            tot = merged(count_lt(cand), (ph + i) % 2)
            take = tot <= k - 1
            return jnp.where(take, cand, res), jnp.where(take, tot, below)

        res, below = lax.fori_loop(0, 15, l1_iter, (zero, zero))
        kk = k - below

        lo = res
        hi = res + jnp.int32(1 << 15)

        def c_iter(i, off):
            o = off
            for j in range(4):
                x = keys[pl.ds((i * 4 + j) * 16, 16)]
                m = (x >= lo) & (x < hi)
                plsc.store_compressed(keys.at[pl.ds(o, 16)], x, mask=m)
                o = o + plsc.all_reduce_population_count(m)[0]
            return o

        off = lax.fori_loop(0, _NV // 4, c_iter, jnp.int32(0))
        plsc.store_scatter(keys, [off + lax.iota(jnp.int32, 16)],
                           jnp.full((16,), _SENT, jnp.int32))
        ntrip = (off + 15) // 16

        def l2_iter(i, res):
            cand = res | (jnp.int32(1) << (jnp.int32(14) - i))
            tot = merged(count_lt_dyn(cand, ntrip), (ph + 1 + i) % 2)
            return jnp.where(tot <= kk - 1, cand, res)

        thr = lax.fori_loop(0, 15, l2_iter, res)

        fzero = jnp.zeros((16,), jnp.float32)
        _CH3 = 16384

        def s3_chunk(j, carry):
            inter, ignp2, allp2, kept = carry
            off = base + j * _CH3
            pltpu.sync_copy(bits_hbm.at[pl.ds(off, _CH3)],
                            keys.at[pl.ds(0, _CH3)])
            pltpu.sync_copy(p_hbm.at[pl.ds(off, _CH3)],
                            keys.at[pl.ds(_CH3, _CH3)])

            def body(i, carry):
                inter, ignp2, allp2, kept = carry
                x = keys[pl.ds(i * 16, 16)]
                pv = plsc.bitcast(keys[pl.ds(_CH3 + i * 16, 16)], jnp.float32)
                fgm = x != jnp.int32(_SENT)
                keptm = fgm & (x <= thr)
                ignm = fgm & (x > thr)
                p2 = pv * pv
                inter = inter + jnp.where(keptm, pv, fzero)
                ignp2 = ignp2 + jnp.where(ignm, p2, fzero)
                allp2 = allp2 + p2
                kept = kept + jnp.where(keptm, 1, 0)
                return inter, ignp2, allp2, kept

            return lax.fori_loop(0, _CH3 // 16, body, carry)

        inter, ignp2, allp2, kept = lax.fori_loop(
            0, _Q // _CH3, s3_chunk, (fzero, fzero, fzero, zero))

        lane = lax.iota(jnp.int32, 16)
        bi = plsc.bitcast(jnp.broadcast_to(jnp.sum(inter), (16,)), jnp.int32)
        bg = plsc.bitcast(jnp.broadcast_to(jnp.sum(ignp2), (16,)), jnp.int32)
        ba = plsc.bitcast(jnp.broadcast_to(jnp.sum(allp2), (16,)), jnp.int32)
        bk = jnp.broadcast_to(jnp.sum(kept), (16,))
        row = jnp.where(lane == 0, bi,
                        jnp.where(lane == 1, bg,
                                  jnp.where(lane == 2, ba, bk)))
        r0, r1, r2, r3 = exchange(row, ph % 2)
        ftot = (plsc.bitcast(r0, jnp.float32) + plsc.bitcast(r1, jnp.float32)
                + plsc.bitcast(r2, jnp.float32) + plsc.bitcast(r3, jnp.float32))
        itot = r0 + r1 + r2 + r3
        inter_t = jnp.broadcast_to(ftot[0], (16,))
        ignp2_t = jnp.broadcast_to(ftot[1], (16,))
        allp2_t = jnp.broadcast_to(ftot[2], (16,))
        kept_t = jnp.broadcast_to(itot[3], (16,)).astype(jnp.float32)
        drow[...] = ((2.0 * inter_t + _SMOOTH)
                     / (allp2_t - ignp2_t + kept_t + _SMOOTH))

        @pl.when(quarter == 0)
        def _():
            pltpu.sync_copy(drow, dice_hbm.at[pl.ds(sample * 16, 16)])


_EPS_CACHE = {}


def _eps(B):
    if B not in _EPS_CACHE:
        eps_key = jax.random.fold_in(jax.random.key(1), 7)
        _EPS_CACHE[B] = (
            jax.random.uniform(eps_key, (B, 262144), dtype=jnp.float32) * 1e-06
        ).reshape(B, 2048, 128)
    return _EPS_CACHE[B]


@jax.jit
def kernel(logits, target):
    B = logits.shape[0]
    lg = logits.reshape(B, 2, 2048, 128)
    tg = target.reshape(B, 2048, 128)
    eps = _eps(B)
    bits, p, nfg = pl.pallas_call(
        _prep_kernel,
        grid=(B,),
        in_specs=[
            pl.BlockSpec((1, 2, 2048, 128), lambda i: (i, 0, 0, 0)),
            pl.BlockSpec((1, 2048, 128), lambda i: (i, 0, 0)),
            pl.BlockSpec((1, 2048, 128), lambda i: (i, 0, 0)),
        ],
        out_specs=[
            pl.BlockSpec((1, 2048, 128), lambda i: (i, 0, 0)),
            pl.BlockSpec((1, 2048, 128), lambda i: (i, 0, 0)),
            pl.BlockSpec((1, 8, 128), lambda i: (i, 0, 0)),
        ],
        out_shape=[
            jax.ShapeDtypeStruct((B, 2048, 128), jnp.int32),
            jax.ShapeDtypeStruct((B, 2048, 128), jnp.float32),
            jax.ShapeDtypeStruct((B, 8, 128), jnp.int32),
        ],
    )(lg, tg, eps)

    p_bits = lax.bitcast_convert_type(p, jnp.int32)
    dice, _ = _select_kernel(bits.reshape(B * _N), p_bits.reshape(B * _N),
                             nfg.reshape(B * 1024))
    return 1.0 - jnp.mean(dice.reshape(B, 16)[:, 0])


# speedup vs baseline: 1.0974x; 1.0974x over previous
import functools

import jax
import jax.numpy as jnp
from jax import lax
from jax.experimental import pallas as pl
from jax.experimental.pallas import tpu as pltpu
from jax.experimental.pallas import tpu_sc as plsc

_SMOOTH = 1e-05
_SENT = 0x7FFFFFFF
_N = 262144
_Q = _N // 4
_NV = _Q // 16


def _prep_kernel(logits_ref, target_ref, eps_ref, bits_ref, p_ref, nfg_ref):
    l = logits_ref[0]
    d = l[1] - l[0]
    p = 1.0 / (1.0 + jnp.exp(-d))
    fg = target_ref[0] == 1
    tp = p * (jnp.where(fg, 1.0, 0.0) + eps_ref[0])
    bits = lax.bitcast_convert_type(tp, jnp.int32)
    bits_ref[0] = jnp.where(fg, bits, jnp.int32(_SENT))
    p_ref[0] = p
    nfg_ref[0] = jnp.full((8, 128), jnp.sum(fg.astype(jnp.int32)),
                          dtype=jnp.int32)


def _dice_kernel(bits_ref, p_ref, thr_ref, out_ref):
    bits = bits_ref[0]
    p = p_ref[0]
    thr = thr_ref[0, 0, 0]
    fg = bits != jnp.int32(_SENT)
    kept = fg & (bits <= thr)
    ign = fg & (bits > thr)
    inter = jnp.sum(jnp.where(kept, p, 0.0))
    p2 = p * p
    ssp = jnp.sum(p2) - jnp.sum(jnp.where(ign, p2, 0.0))
    sst = jnp.sum(jnp.where(kept, 1.0, 0.0))
    dice = (2.0 * inter + _SMOOTH) / (ssp + sst + _SMOOTH)
    out_ref[0] = jnp.full((8, 128), dice, dtype=jnp.float32)


_sc_mesh = plsc.VectorSubcoreMesh(core_axis_name="c", subcore_axis_name="s")


@functools.partial(
    pl.kernel,
    mesh=_sc_mesh,
    compiler_params=pltpu.CompilerParams(needs_layout_passes=False),
    out_type=[
        jax.ShapeDtypeStruct((16 * 16,), jnp.int32),
        jax.ShapeDtypeStruct((64, 16), jnp.int32),
    ],
    scratch_types=[
        pltpu.VMEM((_Q + 16,), jnp.int32),
        pltpu.VMEM((16,), jnp.int32),
        pltpu.VMEM((4, 16), jnp.int32),
    ],
)
def _select_kernel(bits_hbm, nfg_hbm, thr_hbm, cx_hbm, keys, mine, part):
    c = lax.axis_index("c")
    s = lax.axis_index("s")
    quarter = s % 4
    g4 = (s // 4) * 4
    zero = jnp.zeros((16,), jnp.int32)

    def merged(cnt_vec, slot):
        mine[...] = cnt_vec
        pltpu.sync_copy(mine, cx_hbm.at[slot * 32 + c * 16 + s])
        plsc.subcore_barrier()
        pltpu.sync_copy(cx_hbm.at[pl.ds(slot * 32 + c * 16 + g4, 4)], part)
        return part[0] + part[1] + part[2] + part[3]

    def count_lt(cand_vec):
        @plsc.parallel_loop(0, _NV, unroll=8, carry=zero)
        def acc(i, a):
            x = keys[pl.ds(i * 16, 16)]
            return a + jnp.where(x < cand_vec, 1, 0)
        return jnp.broadcast_to(jnp.sum(acc), (16,))

    def count_lt_dyn(cand_vec, ntrip):
        def body(i, acc):
            x = keys[pl.ds(i * 16, 16)]
            return acc + jnp.where(x < cand_vec, 1, 0)
        acc = lax.fori_loop(0, ntrip, body, zero)
        return jnp.broadcast_to(jnp.sum(acc), (16,))

    for ph in range(2):
        sample = ph * 8 + c * 4 + s // 4
        base = sample * _N + quarter * _Q
        pltpu.sync_copy(bits_hbm.at[pl.ds(base, _Q)], keys.at[pl.ds(0, _Q)])
        pltpu.sync_copy(nfg_hbm.at[pl.ds(sample * 1024, 16)], mine)
        n_fg = mine[...]
        k = jnp.maximum(1, n_fg >> 1)

        def l1_iter(i, carry):
            res, below = carry
            cand = res | (jnp.int32(1) << (jnp.int32(29) - i))
            tot = merged(count_lt(cand), i % 2)
            take = tot <= k - 1
            return jnp.where(take, cand, res), jnp.where(take, tot, below)

        res, below = lax.fori_loop(0, 15, l1_iter, (zero, zero))
        kk = k - below

        lo = res
        hi = res + jnp.int32(1 << 15)

        def c_iter(i, off):
            o = off
            for j in range(4):
                x = keys[pl.ds((i * 4 + j) * 16, 16)]
                m = (x >= lo) & (x < hi)
                plsc.store_compressed(keys.at[pl.ds(o, 16)], x, mask=m)
                o = o + plsc.all_reduce_population_count(m)[0]
            return o

        off = lax.fori_loop(0, _NV // 4, c_iter, jnp.int32(0))
        plsc.store_scatter(keys, [off + lax.iota(jnp.int32, 16)],
                           jnp.full((16,), _SENT, jnp.int32))
        ntrip = (off + 15) // 16

        def l2_iter(i, res):
            cand = res | (jnp.int32(1) << (jnp.int32(14) - i))
            tot = merged(count_lt_dyn(cand, ntrip), (i + 1) % 2)
            return jnp.where(tot <= kk - 1, cand, res)

        res = lax.fori_loop(0, 15, l2_iter, res)

        mine[...] = res

        @pl.when(quarter == 0)
        def _():
            pltpu.sync_copy(mine, thr_hbm.at[pl.ds(sample * 16, 16)])


_EPS_CACHE = {}


def _eps(B):
    if B not in _EPS_CACHE:
        eps_key = jax.random.fold_in(jax.random.key(1), 7)
        _EPS_CACHE[B] = (
            jax.random.uniform(eps_key, (B, 262144), dtype=jnp.float32) * 1e-06
        ).reshape(B, 2048, 128)
    return _EPS_CACHE[B]


@jax.jit
def kernel(logits, target):
    B = logits.shape[0]
    lg = logits.reshape(B, 2, 2048, 128)
    tg = target.reshape(B, 2048, 128)
    eps = _eps(B)
    bits, p, nfg = pl.pallas_call(
        _prep_kernel,
        grid=(B,),
        in_specs=[
            pl.BlockSpec((1, 2, 2048, 128), lambda i: (i, 0, 0, 0)),
            pl.BlockSpec((1, 2048, 128), lambda i: (i, 0, 0)),
            pl.BlockSpec((1, 2048, 128), lambda i: (i, 0, 0)),
        ],
        out_specs=[
            pl.BlockSpec((1, 2048, 128), lambda i: (i, 0, 0)),
            pl.BlockSpec((1, 2048, 128), lambda i: (i, 0, 0)),
            pl.BlockSpec((1, 8, 128), lambda i: (i, 0, 0)),
        ],
        out_shape=[
            jax.ShapeDtypeStruct((B, 2048, 128), jnp.int32),
            jax.ShapeDtypeStruct((B, 2048, 128), jnp.float32),
            jax.ShapeDtypeStruct((B, 8, 128), jnp.int32),
        ],
    )(lg, tg, eps)

    thr, _ = _select_kernel(bits.reshape(B * _N), nfg.reshape(B * 1024))

    dice = pl.pallas_call(
        _dice_kernel,
        grid=(B,),
        in_specs=[
            pl.BlockSpec((1, 2048, 128), lambda i: (i, 0, 0)),
            pl.BlockSpec((1, 2048, 128), lambda i: (i, 0, 0)),
            pl.BlockSpec((1, 1, 16), lambda i: (i, 0, 0)),
        ],
        out_specs=pl.BlockSpec((1, 8, 128), lambda i: (i, 0, 0)),
        out_shape=jax.ShapeDtypeStruct((B, 8, 128), jnp.float32),
    )(bits, p, thr.reshape(B, 1, 16))
    return 1.0 - jnp.mean(dice[:, 0, 0])
